# baseline (speedup 1.0000x reference)
import functools
import math

import jax
import jax.numpy as jnp
from jax.experimental import pallas as pl


def _affine_body(use_prelu, a_ref, w_ref, b_ref, al_ref, o_ref):
    acc = jax.lax.dot_general(
        a_ref[...], w_ref[...], (((1,), (1,)), ((), ())),
        preferred_element_type=jnp.float32)
    acc = acc + b_ref[...]
    if use_prelu:
        al = al_ref[0, 0]
        acc = jnp.where(acc >= 0, acc, al * acc)
    o_ref[...] = acc


def _affine(a, W, b, alpha=None, block_m=2048):
    M, F = a.shape
    O = W.shape[0]
    if b is None:
        b = jnp.zeros((O,), jnp.float32)
    use_prelu = alpha is not None
    if alpha is None:
        alpha = jnp.float32(0.0)
    al = jnp.asarray(alpha, jnp.float32).reshape(1, 1)
    grid = (pl.cdiv(M, block_m),)
    return pl.pallas_call(
        functools.partial(_affine_body, use_prelu),
        grid=grid,
        in_specs=[
            pl.BlockSpec((block_m, F), lambda i: (i, 0)),
            pl.BlockSpec((O, F), lambda i: (0, 0)),
            pl.BlockSpec((1, O), lambda i: (0, 0)),
            pl.BlockSpec((1, 1), lambda i: (0, 0)),
        ],
        out_specs=pl.BlockSpec((block_m, O), lambda i: (i, 0)),
        out_shape=jax.ShapeDtypeStruct((M, O), jnp.float32),
    )(a, W, b.reshape(1, O), al)


def _prelu(x, a):
    return jnp.where(x >= 0, x, a * x)


def _spatial_agg(p, x, mask, src, dst, V, inv_cnt, n):
    in_ch = x.shape[1]
    g = _affine(x, p['fgW'], p['fgb'], p['a3'])
    gpr = jnp.mean(g, axis=0)

    f1W, f1b = p['f1W'], p['f1b']
    W1x = f1W[:, :in_ch + 30]
    W1g = f1W[:, in_ch + 30:in_ch + 33]
    b1_eff = f1b + gpr @ W1g.T
    xm = jnp.concatenate([x, mask], axis=1)
    U = _affine(xm, W1x, b1_eff)

    msg = _prelu(jnp.take(U, src, axis=0) + V, p['a1'])
    ssum = jax.ops.segment_sum(msg, dst, num_segments=n)
    agg = ssum * inv_cnt

    f2W, f2b = p['f2W'], p['f2b']
    W2xma = f2W[:, :in_ch + 45]
    W2g = f2W[:, in_ch + 45:in_ch + 48]
    b2_eff = f2b + gpr @ W2g.T
    cat = jnp.concatenate([x, mask, agg], axis=1)
    return _affine(cat, W2xma, b2_eff, p['a2'])


def _edge_V(p, rel):
    e = _affine(rel, p['feW'], p['feb'], p['a4'])
    in_ch = p['f1W'].shape[1] - 48
    W1e = p['f1W'][:, in_ch + 33:]
    return _affine(e, W1e, None)


def _readout(p, inpt, x_query, x_context, k=10, n_heads=5, n_latent=15):
    Q = x_query.shape[0]
    d2 = jnp.sum(jnp.square(x_query[:, None, :] - x_context[None, :, :]), axis=-1)
    _, nbr = jax.lax.top_k(-d2, k)
    rel = (x_query[:, None, :] - x_context[nbr]).reshape(Q * k, 3)
    e = _affine(rel, p['feW'], p['feb'], p['a3'])
    cat = jnp.concatenate([inpt[nbr].reshape(Q * k, -1), e], axis=1)
    ctx = _affine(cat, p['fcW'], p['fcb']).reshape(Q, k, n_heads, n_latent)
    val = _affine(cat, p['fvW'], p['fvb']).reshape(Q, k, n_heads, n_latent)
    alpha = _prelu(jnp.sum(p['pv'] * ctx, axis=-1) / math.sqrt(n_latent), p['a1'])
    alpha = jax.nn.softmax(alpha, axis=1)
    agg = jnp.sum(alpha[..., None] * val, axis=1)
    m = jnp.mean(agg, axis=1)
    h = _affine(m, p['p1W'], p['p1b'], p['a2'])
    return _affine(h, p['p2W'], p['p2b'])


def _embed(p, x):
    h = _affine(x, p['W1'], p['b1'], p['a1'])
    return _affine(h, p['W2'], p['b2'], p['a2'])


def _run(x, mask, x_query, A_edges, A_edges_c_1, merged_nodes, batch,
         batch_query, n_nodes, params):
    n = x.shape[0]
    pos = merged_nodes

    def edge_prep(edges):
        src, dst = edges[0], edges[1]
        rel = jnp.take(pos, dst, axis=0) - jnp.take(pos, src, axis=0)
        cnt = jax.ops.segment_sum(jnp.ones(dst.shape, jnp.float32), dst,
                                  num_segments=n)
        inv = (1.0 / jnp.maximum(cnt, 1.0))[:, None]
        return src, dst, rel, inv

    src_a, dst_a, rel_a, inv_a = edge_prep(A_edges)
    src_c, dst_c, rel_c, inv_c = edge_prep(A_edges_c_1)

    xe = _embed(params['embed'], x)
    me = _embed(params['embed_mask'], mask)

    def layer(name, h, which):
        p = params[name]
        if which == 'a':
            V = _edge_V(p, rel_a)
            return _spatial_agg(p, h, me, src_a, dst_a, V, inv_a, n)
        V = _edge_V(p, rel_c)
        return _spatial_agg(p, h, me, src_c, dst_c, V, inv_c, n)

    out_1 = layer('sa1', xe, 'a')
    out = layer('sa2', out_1, 'a')
    out = layer('sa3', out, 'c') + out_1
    out_2 = layer('sa4', out, 'a')
    out = layer('sa5', out_2, 'a')
    out = layer('sa6', out, 'c') + out_2
    out_3 = layer('sa7', out, 'a')
    out = layer('sa8', out_3, 'a')
    out = layer('sa9', out, 'c') + out_3
    out = layer('sa10', out, 'a')
    return _readout(params['ro'], out, x_query, merged_nodes)


def kernel(x, mask, x_query, A_edges, A_edges_c_1, merged_nodes, batch,
           batch_query, n_nodes, params):
    return _run(x, mask, x_query, A_edges, A_edges_c_1, merged_nodes, batch,
                batch_query, n_nodes, params)
